# baseline (speedup 1.0000x reference)
import jax
import jax.numpy as jnp
from jax.experimental import pallas as pl

_ROWS, _COLS = 16384, 4096
_BLOCK = 1024
_CBLK = 2048


def _copy_patch_kernel(x_ref, o_ref):
    o_ref[...] = x_ref[...]

    @pl.when((pl.program_id(0) == 0) & (pl.program_id(1) == 0))
    def _patch():
        tile = o_ref[0:8, 0:128]
        r = jax.lax.broadcasted_iota(jnp.int32, (8, 128), 0)
        c = jax.lax.broadcasted_iota(jnp.int32, (8, 128), 1)
        tile = jnp.where((r == 0) & (c == 0), 10.0, tile)
        tile = jnp.where((r == 0) & (c == 2), 30.0, tile)
        tile = jnp.where((r == 1) & (c == 0), 40.0, tile)
        tile = jnp.where((r == 1) & (c == 1), 20.0, tile)
        o_ref[0:8, 0:128] = tile


def kernel(x):
    return pl.pallas_call(
        _copy_patch_kernel,
        grid=(_ROWS // _BLOCK, _COLS // _CBLK),
        in_specs=[pl.BlockSpec((_BLOCK, _CBLK), lambda i, j: (i, j))],
        out_specs=pl.BlockSpec((_BLOCK, _CBLK), lambda i, j: (i, j)),
        out_shape=jax.ShapeDtypeStruct((_ROWS, _COLS), jnp.float32),
    )(x)
